# baseline (speedup 1.0000x reference)
import jax
import jax.numpy as jnp
from jax.experimental import pallas as pl

_B, _C, _H, _W = 16, 384, 56, 56
_PS = 3
_STRIDE = 26
_NP = 9


_BB = 16


_EYE9 = None


def _crop_kernel(xt_ref, out_ref):
    nb = pl.program_id(1)
    off = 2 * nb
    eye = jnp.eye(_PS * _PS, dtype=jnp.float32)
    for mb in range(3):
        for bb in range(_BB):
            rows = [xt_ref[bb, off + p // _PS, _STRIDE * mb + p % _PS, :]
                    for p in range(_PS * _PS)]
            m = jnp.stack(rows, axis=0)
            t = jax.lax.dot_general(
                m, eye, (((0,), (0,)), ((), ())),
                precision=jax.lax.Precision.HIGHEST)
            out_ref[bb, _PS * nb + mb, :, :] = t


@jax.jit
def kernel(x):
    xt = jnp.transpose(x, (0, 2, 3, 1))
    out5 = pl.pallas_call(
        _crop_kernel,
        grid=(_B // _BB, _PS),
        in_specs=[
            pl.BlockSpec(
                (_BB, 8, _W, _C),
                lambda b, nb: (b, 3 * nb, 0, 0),
            ),
        ],
        out_specs=pl.BlockSpec(
            (_BB, _NP, _C, _PS * _PS),
            lambda b, nb: (b, 0, 0, 0),
        ),
        out_shape=jax.ShapeDtypeStruct((_B, _NP, _C, _PS * _PS), jnp.float32),
    )(xt)
    return out5.reshape(_B, _NP, _C * _PS * _PS)


# speedup vs baseline: 1.3210x; 1.3210x over previous
import jax
import jax.numpy as jnp
from jax.experimental import pallas as pl

_B, _C, _H, _W = 16, 384, 56, 56
_PS = 3
_STRIDE = 26
_NP = 9


_BB = 16


def _crop_kernel(xt_ref, out_ref):
    nb = pl.program_id(1)
    mb = pl.program_id(2)
    roff = 2 * nb
    woff = 2 * mb
    for pr in range(_PS):
        for pc in range(_PS):
            out_ref[:, _PS * nb + mb, _PS * pr + pc, :] = (
                xt_ref[:, roff + pr, woff + pc, :]
            )


@jax.jit
def kernel(x):
    xt = jnp.transpose(x, (0, 2, 3, 1))
    out5 = pl.pallas_call(
        _crop_kernel,
        grid=(_B // _BB, _PS, _PS),
        in_specs=[
            pl.BlockSpec(
                (_BB, 8, 8, _C),
                lambda b, nb, mb: (b, 3 * nb, 3 * mb, 0),
            ),
        ],
        out_specs=pl.BlockSpec(
            (_BB, _NP, _PS * _PS, _C),
            lambda b, nb, mb: (b, 0, 0, 0),
        ),
        out_shape=jax.ShapeDtypeStruct((_B, _NP, _PS * _PS, _C), jnp.float32),
    )(xt)
    return jnp.transpose(out5, (0, 1, 3, 2)).reshape(_B, _NP, _C * _PS * _PS)
